# baseline (speedup 1.0000x reference)
import functools

import jax
import jax.numpy as jnp
import numpy as np
from jax import lax
from jax.experimental import pallas as pl
from jax.experimental.pallas import tpu as pltpu
from jax.experimental.pallas import tpu_sc as plsc

NUMANGLE = 100
NUMRHO = 100
RPAD = 112


def _rho_table(H, W):
    irho = float(int(np.sqrt(H * H + W * W) + 1)) / float(NUMRHO - 1)
    itheta = np.pi / NUMANGLE
    angles = jnp.arange(NUMANGLE, dtype=jnp.float32) * itheta
    tabCos = jnp.cos(angles) / irho
    tabSin = jnp.sin(angles) / irho
    xs = jnp.arange(W, dtype=jnp.float32) - (W // 2)
    ys = jnp.arange(H, dtype=jnp.float32) - (H // 2)
    r = jnp.round(xs[None, None, :] * tabCos[:, None, None]
                  + ys[None, :, None] * tabSin[:, None, None]).astype(jnp.int32)
    r = r + NUMRHO // 2
    r = jnp.clip(r, 0, NUMRHO - 1)
    return r.reshape(NUMANGLE, 1, H * W)


def _dht_body(feat_ref, r_ref, out_ref):
    rv = r_ref[0, 0, :]
    onehot = (rv[:, None] == lax.broadcasted_iota(jnp.int32, (rv.shape[0], NUMRHO), 1))
    onehot = onehot.astype(jnp.float32)
    out_ref[0] = jnp.dot(feat_ref[...], onehot,
                         preferred_element_type=jnp.float32)


def _dht_sc(feat_flat, ridx):
    NC, P = feat_flat.shape
    A = ridx.shape[0]
    info = plsc.get_sparse_core_info()
    ncores, nsub = info.num_cores, info.num_subcores
    NW = ncores * nsub
    CPW = NC // NW
    NPB = P // 16

    mesh = plsc.VectorSubcoreMesh(core_axis_name="c", subcore_axis_name="s")

    @functools.partial(
        pl.kernel, mesh=mesh,
        compiler_params=pltpu.CompilerParams(needs_layout_passes=False),
        out_type=jax.ShapeDtypeStruct((A, NC, RPAD), jnp.float32),
        scratch_types=(
            [pltpu.VMEM((CPW, P), jnp.float32),
             pltpu.VMEM((1, P), jnp.int32)]
            + [pltpu.VMEM((RPAD,), jnp.float32) for _ in range(CPW)]
            + [pltpu.SemaphoreType.DMA]
        ),
    )
    def k(feat_hbm, ridx_hbm, out_hbm, feat_v, ridx_v, *accs_sem):
        accs, sem = accs_sem[:CPW], accs_sem[CPW]
        wid = lax.axis_index("s") * ncores + lax.axis_index("c")
        base = wid * CPW
        pltpu.sync_copy(feat_hbm.at[pl.ds(base, CPW)], feat_v)

        zeros = jnp.zeros((16,), jnp.float32)

        def angle_body(a, _):
            pltpu.sync_copy(ridx_hbm.at[a], ridx_v.at[0])
            for ch in range(CPW):
                for j in range(RPAD // 16):
                    accs[ch][pl.ds(j * 16, 16)] = zeros

            @plsc.parallel_loop(0, NPB, unroll=4)
            def pixel_body(pb):
                off = pl.multiple_of(pb * 16, 16)
                idx = ridx_v[0, pl.ds(off, 16)]
                for ch in range(CPW):
                    vals = feat_v[ch, pl.ds(off, 16)]
                    plsc.addupdate_scatter(accs[ch], [idx], vals)
            copies = [pltpu.async_copy(accs[ch], out_hbm.at[a, base + ch], sem)
                      for ch in range(CPW)]
            for cp in copies:
                cp.wait()
            return _

        lax.fori_loop(0, A, angle_body, None)

    return k(feat_flat, ridx)


def _dht_tc(feat_flat, r3, interpret=False):
    NC, P = feat_flat.shape
    A_tc = r3.shape[0]
    return pl.pallas_call(
        _dht_body,
        grid=(A_tc,),
        in_specs=[
            pl.BlockSpec((NC, P), lambda a: (0, 0)),
            pl.BlockSpec((1, 1, P), lambda a: (a, 0, 0)),
        ],
        out_specs=pl.BlockSpec((1, NC, NUMRHO), lambda a: (a, 0, 0)),
        out_shape=jax.ShapeDtypeStruct((A_tc, NC, NUMRHO), jnp.float32),
        interpret=interpret,
    )(feat_flat, r3)


A_SC = 32


@functools.partial(jax.jit, static_argnames=("interpret",))
def kernel(feat, interpret=False):
    N, C, H, W = feat.shape
    P = H * W
    NC = N * C
    feat_flat = feat.reshape(NC, P)
    r = _rho_table(H, W)

    parts = []
    if A_SC > 0:
        feat_perm = feat_flat.reshape(NC, 16, P // 16)
        feat_perm = feat_perm.transpose(0, 2, 1).reshape(NC, P)
        r_sc = r[:A_SC].reshape(A_SC, 16, P // 16)
        r_sc = r_sc.transpose(0, 2, 1).reshape(A_SC, P)
        out_sc = _dht_sc(feat_perm, r_sc)[..., :NUMRHO]
        parts.append(out_sc)
    if A_SC < NUMANGLE:
        parts.append(_dht_tc(feat_flat, r[A_SC:], interpret=interpret))

    out = parts[0] if len(parts) == 1 else jnp.concatenate(parts, axis=0)
    return jnp.transpose(out, (1, 0, 2)).reshape(N, C, NUMANGLE, NUMRHO)


# speedup vs baseline: 1.6012x; 1.6012x over previous
import functools

import jax
import jax.numpy as jnp
import numpy as np
from jax import lax
from jax.experimental import pallas as pl
from jax.experimental.pallas import tpu as pltpu
from jax.experimental.pallas import tpu_sc as plsc

NUMANGLE = 100
NUMRHO = 100
RPAD = 112


def _rho_table(H, W):
    irho = float(int(np.sqrt(H * H + W * W) + 1)) / float(NUMRHO - 1)
    itheta = np.pi / NUMANGLE
    angles = jnp.arange(NUMANGLE, dtype=jnp.float32) * itheta
    tabCos = jnp.cos(angles) / irho
    tabSin = jnp.sin(angles) / irho
    xs = jnp.arange(W, dtype=jnp.float32) - (W // 2)
    ys = jnp.arange(H, dtype=jnp.float32) - (H // 2)
    r = jnp.round(xs[None, None, :] * tabCos[:, None, None]
                  + ys[None, :, None] * tabSin[:, None, None]).astype(jnp.int32)
    r = r + NUMRHO // 2
    r = jnp.clip(r, 0, NUMRHO - 1)
    return r.reshape(NUMANGLE, 1, H * W)


def _dht_body(feat_ref, r_ref, out_ref):
    rv = r_ref[0, 0, :]
    onehot = (rv[:, None] == lax.broadcasted_iota(jnp.int32, (rv.shape[0], NUMRHO), 1))
    onehot = onehot.astype(jnp.float32)
    out_ref[0] = jnp.dot(feat_ref[...], onehot,
                         preferred_element_type=jnp.float32)


def _dht_sc(feat_flat, ridx):
    NC, P = feat_flat.shape
    A = ridx.shape[0]
    info = plsc.get_sparse_core_info()
    ncores, nsub = info.num_cores, info.num_subcores
    NW = ncores * nsub
    CPW = NC // NW
    NPB = P // 16

    mesh = plsc.VectorSubcoreMesh(core_axis_name="c", subcore_axis_name="s")

    @functools.partial(
        pl.kernel, mesh=mesh,
        compiler_params=pltpu.CompilerParams(needs_layout_passes=False),
        cost_estimate=pl.CostEstimate(
            flops=2 * NC * P * A, transcendentals=0,
            bytes_accessed=4 * (NC * P + A * P + A * NC * RPAD)),
        out_type=jax.ShapeDtypeStruct((A, NC, RPAD), jnp.float32),
        scratch_types=(
            [pltpu.VMEM((CPW, P), jnp.float32),
             pltpu.VMEM((1, P), jnp.int32)]
            + [pltpu.VMEM((RPAD,), jnp.float32) for _ in range(CPW)]
            + [pltpu.SemaphoreType.DMA]
        ),
    )
    def k(feat_hbm, ridx_hbm, out_hbm, feat_v, ridx_v, *accs_sem):
        accs, sem = accs_sem[:CPW], accs_sem[CPW]
        wid = lax.axis_index("s") * ncores + lax.axis_index("c")
        base = wid * CPW
        pltpu.sync_copy(feat_hbm.at[pl.ds(base, CPW)], feat_v)

        zeros = jnp.zeros((16,), jnp.float32)

        def angle_body(a, _):
            pltpu.sync_copy(ridx_hbm.at[a], ridx_v.at[0])
            for ch in range(CPW):
                for j in range(RPAD // 16):
                    accs[ch][pl.ds(j * 16, 16)] = zeros

            @plsc.parallel_loop(0, NPB, unroll=4)
            def pixel_body(pb):
                off = pl.multiple_of(pb * 16, 16)
                idx = ridx_v[0, pl.ds(off, 16)]
                for ch in range(CPW):
                    vals = feat_v[ch, pl.ds(off, 16)]
                    plsc.addupdate_scatter(accs[ch], [idx], vals)
            copies = [pltpu.async_copy(accs[ch], out_hbm.at[a, base + ch], sem)
                      for ch in range(CPW)]
            for cp in copies:
                cp.wait()
            return _

        lax.fori_loop(0, A, angle_body, None)

    return k(feat_flat, ridx)


def _dht_tc(feat_flat, r3, interpret=False):
    NC, P = feat_flat.shape
    A_tc = r3.shape[0]
    return pl.pallas_call(
        _dht_body,
        grid=(A_tc,),
        in_specs=[
            pl.BlockSpec((NC, P), lambda a: (0, 0)),
            pl.BlockSpec((1, 1, P), lambda a: (a, 0, 0)),
        ],
        out_specs=pl.BlockSpec((1, NC, NUMRHO), lambda a: (a, 0, 0)),
        out_shape=jax.ShapeDtypeStruct((A_tc, NC, NUMRHO), jnp.float32),
        interpret=interpret,
    )(feat_flat, r3)


A_SC = 16


@functools.partial(jax.jit, static_argnames=("interpret",))
def kernel(feat, interpret=False):
    N, C, H, W = feat.shape
    P = H * W
    NC = N * C
    feat_flat = feat.reshape(NC, P)
    r = _rho_table(H, W)

    parts = []
    if A_SC > 0:
        feat_perm = feat_flat.reshape(NC, 16, P // 16)
        feat_perm = feat_perm.transpose(0, 2, 1).reshape(NC, P)
        r_sc = r[:A_SC].reshape(A_SC, 16, P // 16)
        r_sc = r_sc.transpose(0, 2, 1).reshape(A_SC, P)
        out_sc = _dht_sc(feat_perm, r_sc)[..., :NUMRHO]
        parts.append(out_sc)
    if A_SC < NUMANGLE:
        parts.append(_dht_tc(feat_flat, r[A_SC:], interpret=interpret))

    out = parts[0] if len(parts) == 1 else jnp.concatenate(parts, axis=0)
    return jnp.transpose(out, (1, 0, 2)).reshape(N, C, NUMANGLE, NUMRHO)


# speedup vs baseline: 2.9375x; 1.8346x over previous
import functools

import jax
import jax.numpy as jnp
import numpy as np
from jax import lax
from jax.experimental import pallas as pl
from jax.experimental.pallas import tpu as pltpu
from jax.experimental.pallas import tpu_sc as plsc

NUMANGLE = 100
NUMRHO = 100
RPAD = 112


def _rho_table(H, W):
    irho = float(int(np.sqrt(H * H + W * W) + 1)) / float(NUMRHO - 1)
    itheta = np.pi / NUMANGLE
    angles = jnp.arange(NUMANGLE, dtype=jnp.float32) * itheta
    tabCos = jnp.cos(angles) / irho
    tabSin = jnp.sin(angles) / irho
    xs = jnp.arange(W, dtype=jnp.float32) - (W // 2)
    ys = jnp.arange(H, dtype=jnp.float32) - (H // 2)
    r = jnp.round(xs[None, None, :] * tabCos[:, None, None]
                  + ys[None, :, None] * tabSin[:, None, None]).astype(jnp.int32)
    r = r + NUMRHO // 2
    r = jnp.clip(r, 0, NUMRHO - 1)
    return r.reshape(NUMANGLE, 1, H * W)


AB = 4
KC = 1


def _dht_body(feat_ref, r_ref, out_ref):
    P = r_ref.shape[2]
    Pc = P // KC
    acc = None
    for c in range(KC):
        sl = pl.ds(c * Pc, Pc)
        iot = lax.broadcasted_iota(jnp.int32, (NUMRHO, Pc), 0)
        ohs = [(iot == r_ref[j, 0, sl][None, :]).astype(jnp.bfloat16)
               for j in range(AB)]
        ohT = jnp.concatenate(ohs, axis=0)
        d = lax.dot_general(feat_ref[:, sl], ohT,
                            (((1,), (1,)), ((), ())),
                            preferred_element_type=jnp.float32)
        acc = d if acc is None else acc + d
    out_ref[0] = acc


def _dht_sc(feat_flat, ridx):
    NC, P = feat_flat.shape
    A = ridx.shape[0]
    info = plsc.get_sparse_core_info()
    ncores, nsub = info.num_cores, info.num_subcores
    NW = ncores * nsub
    CPW = NC // NW
    NPB = P // 16

    mesh = plsc.VectorSubcoreMesh(core_axis_name="c", subcore_axis_name="s")

    @functools.partial(
        pl.kernel, mesh=mesh,
        compiler_params=pltpu.CompilerParams(needs_layout_passes=False),
        cost_estimate=pl.CostEstimate(
            flops=2 * NC * P * A, transcendentals=0,
            bytes_accessed=4 * (NC * P + A * P + A * NC * RPAD)),
        out_type=jax.ShapeDtypeStruct((A, NC, RPAD), jnp.float32),
        scratch_types=(
            [pltpu.VMEM((CPW, P), jnp.float32),
             pltpu.VMEM((1, P), jnp.int32)]
            + [pltpu.VMEM((RPAD,), jnp.float32) for _ in range(CPW)]
            + [pltpu.SemaphoreType.DMA]
        ),
    )
    def k(feat_hbm, ridx_hbm, out_hbm, feat_v, ridx_v, *accs_sem):
        accs, sem = accs_sem[:CPW], accs_sem[CPW]
        wid = lax.axis_index("s") * ncores + lax.axis_index("c")
        base = wid * CPW
        pltpu.sync_copy(feat_hbm.at[pl.ds(base, CPW)], feat_v)

        zeros = jnp.zeros((16,), jnp.float32)

        def angle_body(a, _):
            pltpu.sync_copy(ridx_hbm.at[a], ridx_v.at[0])
            for ch in range(CPW):
                for j in range(RPAD // 16):
                    accs[ch][pl.ds(j * 16, 16)] = zeros

            @plsc.parallel_loop(0, NPB, unroll=4)
            def pixel_body(pb):
                off = pl.multiple_of(pb * 16, 16)
                idx = ridx_v[0, pl.ds(off, 16)]
                for ch in range(CPW):
                    vals = feat_v[ch, pl.ds(off, 16)]
                    plsc.addupdate_scatter(accs[ch], [idx], vals)
            copies = [pltpu.async_copy(accs[ch], out_hbm.at[a, base + ch], sem)
                      for ch in range(CPW)]
            for cp in copies:
                cp.wait()
            return _

        lax.fori_loop(0, A, angle_body, None)

    return k(feat_flat, ridx)


def _dht_tc(feat_flat, r3, interpret=False):
    NC, P = feat_flat.shape
    A_tc = r3.shape[0]
    assert A_tc % AB == 0
    out = pl.pallas_call(
        _dht_body,
        grid=(A_tc // AB,),
        in_specs=[
            pl.BlockSpec((NC, P), lambda s: (0, 0)),
            pl.BlockSpec((AB, 1, P), lambda s: (s, 0, 0)),
        ],
        out_specs=pl.BlockSpec((1, NC, AB * NUMRHO), lambda s: (s, 0, 0)),
        out_shape=jax.ShapeDtypeStruct((A_tc // AB, NC, AB * NUMRHO), jnp.float32),
        interpret=interpret,
    )(feat_flat.astype(jnp.bfloat16), r3)
    out = out.reshape(A_tc // AB, NC, AB, NUMRHO)
    return jnp.transpose(out, (0, 2, 1, 3)).reshape(A_tc, NC, NUMRHO)


A_SC = 0


@functools.partial(jax.jit, static_argnames=("interpret",))
def kernel(feat, interpret=False):
    N, C, H, W = feat.shape
    P = H * W
    NC = N * C
    feat_flat = feat.reshape(NC, P)
    r = _rho_table(H, W)

    parts = []
    if A_SC > 0:
        feat_perm = feat_flat.reshape(NC, 16, P // 16)
        feat_perm = feat_perm.transpose(0, 2, 1).reshape(NC, P)
        r_sc = r[:A_SC].reshape(A_SC, 16, P // 16)
        r_sc = r_sc.transpose(0, 2, 1).reshape(A_SC, P)
        out_sc = _dht_sc(feat_perm, r_sc)[..., :NUMRHO]
        parts.append(out_sc)
    if A_SC < NUMANGLE:
        parts.append(_dht_tc(feat_flat, r[A_SC:], interpret=interpret))

    out = parts[0] if len(parts) == 1 else jnp.concatenate(parts, axis=0)
    return jnp.transpose(out, (1, 0, 2)).reshape(N, C, NUMANGLE, NUMRHO)
